# Initial kernel scaffold; baseline (speedup 1.0000x reference)
import jax
import jax.numpy as jnp
from jax.experimental import pallas as pl


def kernel(feature, x, gW1, gb1, gW2, gb2, aW, ab):
    raise NotImplementedError("write your pallas kernel here")


# speedup vs baseline: 1.8066x; 1.8066x over previous
import functools

import jax
import jax.numpy as jnp
import numpy as np
from jax.experimental import pallas as pl
from jax.experimental.pallas import tpu as pltpu

E = 8
K = 2
D = 768
H = 512
T = 8192
EPS = float(np.finfo(float).eps)

BT_GATE = 1024
BT_EXP = 1024


def _gating_body(f_ref, gW1_ref, gb1_ref, gW2_ref, gb2_ref,
                 gates_ref, imp_ref, loss_ref):
    i = pl.program_id(0)
    nsteps = pl.num_programs(0)

    f = f_ref[...]
    h = jnp.maximum(
        jnp.dot(f, gW1_ref[...], preferred_element_type=jnp.float32)
        + gb1_ref[...], 0.0)
    logits = (jnp.dot(h, gW2_ref[...], preferred_element_type=jnp.float32)
              + gb2_ref[...])

    lane = jax.lax.broadcasted_iota(jnp.int32, logits.shape, 1)
    m1 = jnp.max(logits, axis=1, keepdims=True)
    a1 = jnp.argmax(logits, axis=1).astype(jnp.int32)[:, None]
    neg = jnp.full_like(logits, -jnp.inf)
    masked = jnp.where(lane == a1, neg, logits)
    m2 = jnp.max(masked, axis=1, keepdims=True)
    a2 = jnp.argmax(masked, axis=1).astype(jnp.int32)[:, None]
    e2 = jnp.exp(m2 - m1)
    denom = 1.0 + e2
    g1 = 1.0 / denom
    g2 = e2 / denom

    gates = (jnp.where(lane == a1, g1, 0.0)
             + jnp.where(lane == a2, g2, 0.0))
    gates_ref[...] = gates

    imp_blk = jnp.sum(gates, axis=0, keepdims=True)
    load_blk = jnp.sum((gates > 0.0).astype(jnp.float32), axis=0,
                       keepdims=True)
    blk = jnp.concatenate([imp_blk, load_blk], axis=0)

    @pl.when(i == 0)
    def _init():
        imp_ref[...] = blk

    @pl.when(i > 0)
    def _acc():
        imp_ref[...] += blk

    @pl.when(i == nsteps - 1)
    def _loss():
        acc = imp_ref[...]
        mean = jnp.mean(acc, axis=1, keepdims=True)
        var = jnp.sum((acc - mean) ** 2, axis=1, keepdims=True) / (E - 1)
        cv2 = var / (mean * mean + 1e-10)
        loss_ref[...] = (cv2[0:1, 0:1] + cv2[1:2, 0:1]) * 1e-2


def _expert_body(x_ref, aW_ref, ab_ref, gates_ref, y_ref, acc_ref):
    e = pl.program_id(1)

    z = (jnp.dot(x_ref[...], aW_ref[0], preferred_element_type=jnp.float32)
         + ab_ref[0])
    lane = jax.lax.broadcasted_iota(jnp.int32, gates_ref.shape, 1)
    g = jnp.sum(jnp.where(lane == e, gates_ref[...], 0.0), axis=1,
                keepdims=True)
    contrib = g * jnp.exp(z)

    @pl.when(e == 0)
    def _init():
        acc_ref[...] = contrib

    @pl.when(e > 0)
    def _acc():
        acc_ref[...] += contrib

    @pl.when(e == E - 1)
    def _fin():
        acc = acc_ref[...]
        y_ref[...] = jnp.log(jnp.where(acc == 0.0, EPS, acc))


@jax.jit
def kernel(feature, x, gW1, gb1, gW2, gb2, aW, ab):
    n_gate = T // BT_GATE
    gates, imp_load, loss2d = pl.pallas_call(
        _gating_body,
        grid=(n_gate,),
        in_specs=[
            pl.BlockSpec((BT_GATE, D), lambda i: (i, 0)),
            pl.BlockSpec((D, H), lambda i: (0, 0)),
            pl.BlockSpec((H,), lambda i: (0,)),
            pl.BlockSpec((H, E), lambda i: (0, 0)),
            pl.BlockSpec((E,), lambda i: (0,)),
        ],
        out_specs=[
            pl.BlockSpec((BT_GATE, E), lambda i: (i, 0)),
            pl.BlockSpec((2, E), lambda i: (0, 0)),
            pl.BlockSpec((1, 1), lambda i: (0, 0)),
        ],
        out_shape=[
            jax.ShapeDtypeStruct((T, E), jnp.float32),
            jax.ShapeDtypeStruct((2, E), jnp.float32),
            jax.ShapeDtypeStruct((1, 1), jnp.float32),
        ],
        compiler_params=pltpu.CompilerParams(
            dimension_semantics=("arbitrary",)),
    )(feature, gW1, gb1, gW2, gb2)

    n_exp = T // BT_EXP
    y = pl.pallas_call(
        _expert_body,
        grid=(n_exp, E),
        in_specs=[
            pl.BlockSpec((BT_EXP, D), lambda t, e: (t, 0)),
            pl.BlockSpec((1, D, D), lambda t, e: (e, 0, 0)),
            pl.BlockSpec((1, 1, D), lambda t, e: (e, 0, 0)),
            pl.BlockSpec((BT_EXP, E), lambda t, e: (t, 0)),
        ],
        out_specs=pl.BlockSpec((BT_EXP, D), lambda t, e: (t, 0)),
        out_shape=jax.ShapeDtypeStruct((T, D), jnp.float32),
        scratch_shapes=[pltpu.VMEM((BT_EXP, D), jnp.float32)],
        compiler_params=pltpu.CompilerParams(
            dimension_semantics=("parallel", "arbitrary")),
    )(x, aW, ab.reshape(E, 1, D), gates)

    return y, loss2d[0, 0]
